# Initial kernel scaffold; baseline (speedup 1.0000x reference)
import jax
import jax.numpy as jnp
from jax.experimental import pallas as pl


def kernel(first, second, emb1, emb2):
    raise NotImplementedError("write your pallas kernel here")


# speedup vs baseline: 1.4068x; 1.4068x over previous
import functools

import jax
import jax.numpy as jnp
from jax import lax
from jax.experimental import pallas as pl
from jax.experimental.pallas import tpu as pltpu
from jax.experimental.pallas import tpu_sc as plsc

LANES = 16


@functools.lru_cache(maxsize=None)
def _build_sc_call(n_rows: int, emb_dim: int):
    NW = 32
    per_w = n_rows // NW
    chunk = 128
    n_chunks = per_w // chunk
    assert per_w * NW == n_rows and n_chunks * chunk == per_w

    mesh = plsc.VectorSubcoreMesh(core_axis_name="c", subcore_axis_name="s")

    @functools.partial(
        pl.kernel,
        out_type=jax.ShapeDtypeStruct((n_rows, emb_dim), jnp.float32),
        mesh=mesh,
        compiler_params=pltpu.CompilerParams(use_tc_tiling_on_sc=False),
        scratch_types=[
            pltpu.VMEM((per_w,), jnp.int32),
            pltpu.VMEM((per_w,), jnp.int32),
            pltpu.VMEM((chunk, emb_dim), jnp.float32),
            pltpu.VMEM((chunk, emb_dim), jnp.float32),
            pltpu.SemaphoreType.DMA,
            pltpu.SemaphoreType.DMA,
        ],
    )
    def sc_call(idx1_hbm, idx2_hbm, emb1_hbm, emb2_hbm, out_hbm,
                idx1_v, idx2_v, rows1_v, rows2_v, sem1, sem2):
        wid = lax.axis_index("s") * 2 + lax.axis_index("c")
        base = wid * per_w
        pltpu.sync_copy(idx1_hbm.at[pl.ds(base, per_w)], idx1_v)
        pltpu.sync_copy(idx2_hbm.at[pl.ds(base, per_w)], idx2_v)

        def chunk_body(j, carry):
            off = j * chunk
            cp1 = pltpu.async_copy(
                emb1_hbm.at[idx1_v.at[pl.ds(off, chunk)]], rows1_v, sem1)
            cp2 = pltpu.async_copy(
                emb2_hbm.at[idx2_v.at[pl.ds(off, chunk)]], rows2_v, sem2)
            cp1.wait()
            cp2.wait()

            def mul_body(r, c):
                for h in range(emb_dim // LANES):
                    sl = pl.ds(h * LANES, LANES)
                    rows1_v[r, sl] = rows1_v[r, sl] * rows2_v[r, sl]
                return c

            lax.fori_loop(0, chunk, mul_body, 0, unroll=4)
            pltpu.sync_copy(rows1_v, out_hbm.at[pl.ds(base + off, chunk)])
            return carry

        lax.fori_loop(0, n_chunks, chunk_body, 0)

    return sc_call


def kernel(first, second, emb1, emb2):
    b, f = first.shape
    emb_dim = emb1.shape[1]
    n_rows = b * f
    idx1 = first.reshape(n_rows).astype(jnp.int32)
    idx2 = second.reshape(n_rows).astype(jnp.int32)
    sc_call = _build_sc_call(n_rows, emb_dim)
    out = sc_call(idx1, idx2, emb1, emb2)
    return out.reshape(b, f, emb_dim)


# speedup vs baseline: 1.5168x; 1.0782x over previous
import functools

import jax
import jax.numpy as jnp
from jax import lax
from jax.experimental import pallas as pl
from jax.experimental.pallas import tpu as pltpu
from jax.experimental.pallas import tpu_sc as plsc

LANES = 16


@functools.lru_cache(maxsize=None)
def _build_sc_call(n_rows: int, emb_dim: int):
    NW = 32
    per_w = n_rows // NW
    chunk = 128
    blk = 512
    gpb = blk // chunk
    n_blk = per_w // blk
    assert per_w * NW == n_rows and n_blk * blk == per_w and n_blk % 2 == 0

    mesh = plsc.VectorSubcoreMesh(core_axis_name="c", subcore_axis_name="s")

    @functools.partial(
        pl.kernel,
        out_type=jax.ShapeDtypeStruct((n_rows, emb_dim), jnp.float32),
        mesh=mesh,
        compiler_params=pltpu.CompilerParams(use_tc_tiling_on_sc=False),
        scratch_types=[
            pltpu.VMEM((per_w,), jnp.int32),
            pltpu.VMEM((per_w,), jnp.int32),
            pltpu.VMEM((blk, emb_dim), jnp.float32),
            pltpu.VMEM((blk, emb_dim), jnp.float32),
            pltpu.VMEM((blk, emb_dim), jnp.float32),
            pltpu.VMEM((blk, emb_dim), jnp.float32),
            pltpu.VMEM((blk, emb_dim), jnp.float32),
            pltpu.VMEM((blk, emb_dim), jnp.float32),
            pltpu.SemaphoreType.DMA,
            pltpu.SemaphoreType.DMA,
            pltpu.SemaphoreType.DMA,
            pltpu.SemaphoreType.DMA,
        ],
    )
    def sc_call(idx1_hbm, idx2_hbm, emb1_hbm, emb2_hbm, out_hbm,
                idx1_v, idx2_v, r1a, r1b, r2a, r2b, pa, pb,
                sg_a, sg_b, so_a, so_b):
        r1 = (r1a, r1b)
        r2 = (r2a, r2b)
        prod = (pa, pb)
        sg = (sg_a, sg_b)
        so = (so_a, so_b)

        wid = lax.axis_index("s") * 2 + lax.axis_index("c")
        base = wid * per_w
        pltpu.sync_copy(idx1_hbm.at[pl.ds(base, per_w)], idx1_v)
        pltpu.sync_copy(idx2_hbm.at[pl.ds(base, per_w)], idx2_v)

        def fire_gathers(j, slot):
            off = j * blk
            for g in range(gpb):
                o = off + g * chunk
                dst = pl.ds(g * chunk, chunk)
                pltpu.async_copy(
                    emb1_hbm.at[idx1_v.at[pl.ds(o, chunk)]],
                    r1[slot].at[dst], sg[slot])
                pltpu.async_copy(
                    emb2_hbm.at[idx2_v.at[pl.ds(o, chunk)]],
                    r2[slot].at[dst], sg[slot])

        def drain_gathers(slot):
            pltpu.make_async_copy(
                emb1_hbm.at[pl.ds(0, blk)], r1[slot], sg[slot]).wait()
            pltpu.make_async_copy(
                emb1_hbm.at[pl.ds(0, blk)], r2[slot], sg[slot]).wait()

        def drain_out(slot):
            pltpu.make_async_copy(
                prod[slot], out_hbm.at[pl.ds(0, blk)], so[slot]).wait()

        def multiply(slot):
            a, b, p = r1[slot], r2[slot], prod[slot]

            def mul_body(r, c):
                for h in range(emb_dim // LANES):
                    sl = pl.ds(h * LANES, LANES)
                    p[r, sl] = a[r, sl] * b[r, sl]
                return c

            lax.fori_loop(0, blk, mul_body, 0, unroll=8)

        fire_gathers(0, 0)
        fire_gathers(1, 1)

        def step(i, carry):
            for slot in range(2):
                j = 2 * i + slot
                drain_gathers(slot)

                @pl.when(i > 0)
                def _():
                    drain_out(slot)

                multiply(slot)

                @pl.when(j + 2 < n_blk)
                def _():
                    fire_gathers(j + 2, slot)

                pltpu.async_copy(
                    prod[slot], out_hbm.at[pl.ds(base + j * blk, blk)],
                    so[slot])
            return carry

        lax.fori_loop(0, n_blk // 2, step, 0)
        drain_out(0)
        drain_out(1)

    return sc_call


def kernel(first, second, emb1, emb2):
    b, f = first.shape
    emb_dim = emb1.shape[1]
    n_rows = b * f
    idx1 = first.reshape(n_rows).astype(jnp.int32)
    idx2 = second.reshape(n_rows).astype(jnp.int32)
    sc_call = _build_sc_call(n_rows, emb_dim)
    out = sc_call(idx1, idx2, emb1, emb2)
    return out.reshape(b, f, emb_dim)
